# baseline (speedup 1.0000x reference)
import jax
import jax.numpy as jnp
from jax.experimental import pallas as pl
from jax.experimental.pallas import tpu as pltpu

BM = 400


def _body(adj_ref, x_ref, w10_ref, w11_ref, b1_ref, w20_ref, w21_ref, b2_ref,
          o_ref, g_ref, hw_ref):
    p = pl.program_id(0)
    i = pl.program_id(1)
    rows = pl.ds(i * BM, BM)

    @pl.when(p == 0)
    def _():
        y = jnp.dot(adj_ref[...], x_ref[...], preferred_element_type=jnp.float32)
        h = (
            jnp.dot(x_ref[rows, :], w10_ref[...], preferred_element_type=jnp.float32)
            + jnp.dot(y, w11_ref[...], preferred_element_type=jnp.float32)
            + b1_ref[...]
        )
        h = jnp.maximum(h, 0.0)
        g_ref[rows, :] = jnp.dot(h, w21_ref[...], preferred_element_type=jnp.float32)
        hw_ref[rows, :] = (
            jnp.dot(h, w20_ref[...], preferred_element_type=jnp.float32) + b2_ref[...]
        )

    @pl.when(p == 1)
    def _():
        o = hw_ref[rows, :] + jnp.dot(
            adj_ref[...], g_ref[...], preferred_element_type=jnp.float32
        )
        m = jnp.max(o, axis=1, keepdims=True)
        e = jnp.exp(o - m)
        lse = jnp.log(jnp.sum(e, axis=1, keepdims=True))
        o_ref[...] = o - m - lse


@jax.jit
def kernel(x, adj, W1_0, W1_1, b1, W2_0, W2_1, b2):
    n, f_in = x.shape
    hid = W1_0.shape[1]
    c_out = W2_0.shape[1]
    grid = (2, n // BM)
    return pl.pallas_call(
        _body,
        grid=grid,
        in_specs=[
            pl.BlockSpec((BM, n), lambda p, i: (i, 0)),
            pl.BlockSpec((n, f_in), lambda p, i: (0, 0)),
            pl.BlockSpec((f_in, hid), lambda p, i: (0, 0)),
            pl.BlockSpec((f_in, hid), lambda p, i: (0, 0)),
            pl.BlockSpec((1, hid), lambda p, i: (0, 0)),
            pl.BlockSpec((hid, c_out), lambda p, i: (0, 0)),
            pl.BlockSpec((hid, c_out), lambda p, i: (0, 0)),
            pl.BlockSpec((1, c_out), lambda p, i: (0, 0)),
        ],
        out_specs=pl.BlockSpec((BM, c_out), lambda p, i: (jnp.where(p == 0, 0, i), 0)),
        out_shape=jax.ShapeDtypeStruct((n, c_out), jnp.float32),
        scratch_shapes=[
            pltpu.VMEM((n, c_out), jnp.float32),
            pltpu.VMEM((n, c_out), jnp.float32),
        ],
    )(adj, x, W1_0, W1_1, b1.reshape(1, hid), W2_0, W2_1, b2.reshape(1, c_out))


# speedup vs baseline: 1.0060x; 1.0060x over previous
import jax
import jax.numpy as jnp
from jax.experimental import pallas as pl
from jax.experimental.pallas import tpu as pltpu

BM = 400


def _body(adj_ref, x_ref, w10_ref, w11_ref, b1_ref, w20_ref, w21_ref, b2_ref,
          o_ref, g_ref, hw_ref):
    p = pl.program_id(0)
    i = pl.program_id(1)
    ni = pl.num_programs(1)

    @pl.when(p == 0)
    def _():
        rows = pl.ds(i * BM, BM)
        y = jnp.dot(adj_ref[...], x_ref[...], preferred_element_type=jnp.float32)
        h = (
            jnp.dot(x_ref[rows, :], w10_ref[...], preferred_element_type=jnp.float32)
            + jnp.dot(y, w11_ref[...], preferred_element_type=jnp.float32)
            + b1_ref[...]
        )
        h = jnp.maximum(h, 0.0)
        g_ref[rows, :] = jnp.dot(h, w21_ref[...], preferred_element_type=jnp.float32)
        hw_ref[rows, :] = (
            jnp.dot(h, w20_ref[...], preferred_element_type=jnp.float32) + b2_ref[...]
        )

    @pl.when(p == 1)
    def _():
        rows = pl.ds((ni - 1 - i) * BM, BM)
        o = hw_ref[rows, :] + jnp.dot(
            adj_ref[...], g_ref[...], preferred_element_type=jnp.float32
        )
        m = jnp.max(o, axis=1, keepdims=True)
        e = jnp.exp(o - m)
        lse = jnp.log(jnp.sum(e, axis=1, keepdims=True))
        o_ref[...] = o - m - lse


@jax.jit
def kernel(x, adj, W1_0, W1_1, b1, W2_0, W2_1, b2):
    n, f_in = x.shape
    hid = W1_0.shape[1]
    c_out = W2_0.shape[1]
    ni = n // BM
    grid = (2, ni)
    serp = lambda p, i: jnp.where(p == 0, i, ni - 1 - i)
    out_idx = lambda p, i: (jnp.where(p == 0, ni - 1, ni - 1 - i), 0)
    return pl.pallas_call(
        _body,
        grid=grid,
        in_specs=[
            pl.BlockSpec((BM, n), lambda p, i: (serp(p, i), 0)),
            pl.BlockSpec((n, f_in), lambda p, i: (0, 0)),
            pl.BlockSpec((f_in, hid), lambda p, i: (0, 0)),
            pl.BlockSpec((f_in, hid), lambda p, i: (0, 0)),
            pl.BlockSpec((1, hid), lambda p, i: (0, 0)),
            pl.BlockSpec((hid, c_out), lambda p, i: (0, 0)),
            pl.BlockSpec((hid, c_out), lambda p, i: (0, 0)),
            pl.BlockSpec((1, c_out), lambda p, i: (0, 0)),
        ],
        out_specs=pl.BlockSpec((BM, c_out), out_idx),
        out_shape=jax.ShapeDtypeStruct((n, c_out), jnp.float32),
        scratch_shapes=[
            pltpu.VMEM((n, c_out), jnp.float32),
            pltpu.VMEM((n, c_out), jnp.float32),
        ],
    )(adj, x, W1_0, W1_1, b1.reshape(1, hid), W2_0, W2_1, b2.reshape(1, c_out))
